# Initial kernel scaffold; baseline (speedup 1.0000x reference)
import jax
import jax.numpy as jnp
from jax.experimental import pallas as pl


def kernel(data, mask, rope_cos, rope_sin, wq, wk, wv, wo, w1, w2, w3, gate_w, attn_norm_w, ffn_norm_w, w1_la, w1_lb, w3_la, w3_lb, w2_la, w2_lb):
    raise NotImplementedError("write your pallas kernel here")


# speedup vs baseline: 2.3172x; 2.3172x over previous
import functools

import jax
import jax.numpy as jnp
from jax import lax
from jax.experimental import pallas as pl
from jax.experimental.pallas import tpu as pltpu

B, S, D = 1, 2048, 2048
NH, NKV = 16, 8
HD = D // NH
DFF = 5632
E, K = 8, 2
R = 16
SCALE = 32.0 / 16.0
EPS = 1e-5

TS_A = 256
TS_Q = 256
TS_C = 256
TS_E = 256
F_E = 512
NJ = DFF // F_E


def _qkv_body(x_ref, nw_ref, wqkv_ref, cc_ref, ss2_ref, p_ref,
              q_ref, k_ref, v_ref):
    x = x_ref[...]
    nw = nw_ref[...]
    var = jnp.mean(x * x, axis=-1, keepdims=True)
    h = x * lax.rsqrt(var + EPS) * nw
    qkv = jnp.dot(h, wqkv_ref[...], preferred_element_type=jnp.float32)
    cc = cc_ref[...]
    ss2 = ss2_ref[...]
    p = p_ref[...]
    for hh in range(NH):
        qh = qkv[:, hh * HD:(hh + 1) * HD]
        sw = jnp.dot(qh, p, preferred_element_type=jnp.float32)
        q_ref[:, hh * HD:(hh + 1) * HD] = qh * cc + sw * ss2
    for hh in range(NKV):
        base = NH * HD + hh * HD
        kh = qkv[:, base:base + HD]
        sw = jnp.dot(kh, p, preferred_element_type=jnp.float32)
        k_ref[:, hh * HD:(hh + 1) * HD] = kh * cc + sw * ss2
    v_ref[...] = qkv[:, (NH + NKV) * HD:]


def _attn_body(q_ref, k_ref, v_ref, o_ref):
    i = pl.program_id(1)
    q = q_ref[...]
    k = k_ref[...]
    scores = lax.dot_general(q, k, (((1,), (1,)), ((), ())),
                             preferred_element_type=jnp.float32)
    scores = scores * (1.0 / (HD ** 0.5))
    row = lax.broadcasted_iota(jnp.int32, scores.shape, 0) + i * TS_Q
    col = lax.broadcasted_iota(jnp.int32, scores.shape, 1)
    scores = jnp.where(col <= row, scores, -1e9)
    m = jnp.max(scores, axis=-1, keepdims=True)
    p = jnp.exp(scores - m)
    att = p / jnp.sum(p, axis=-1, keepdims=True)
    o_ref[...] = lax.dot_general(att, v_ref[...], (((1,), (0,)), ((), ())),
                                 preferred_element_type=jnp.float32)


def _proj_router_body(attn_ref, data_ref, wo_ref, nw_ref, gw_ref,
                      la1_ref, la3_ref,
                      d2_ref, sn_ref, ew_ref, a1_ref, a3_ref):
    d2 = jnp.dot(attn_ref[...], wo_ref[...],
                 preferred_element_type=jnp.float32) + data_ref[...]
    d2_ref[...] = d2
    var = jnp.mean(d2 * d2, axis=-1, keepdims=True)
    sn = d2 * lax.rsqrt(var + EPS) * nw_ref[...]
    sn_ref[...] = sn
    a1_ref[...] = jnp.dot(sn, la1_ref[...], preferred_element_type=jnp.float32)
    a3_ref[...] = jnp.dot(sn, la3_ref[...], preferred_element_type=jnp.float32)
    logits = jnp.dot(sn, gw_ref[...], preferred_element_type=jnp.float32)
    mx = jnp.max(logits, axis=-1, keepdims=True)
    pexp = jnp.exp(logits - mx)
    prob = pexp / jnp.sum(pexp, axis=-1, keepdims=True)
    eidx = lax.broadcasted_iota(jnp.int32, prob.shape, 1)
    m1 = jnp.max(prob, axis=-1, keepdims=True)
    i1 = jnp.min(jnp.where(prob == m1, eidx, E), axis=-1, keepdims=True)
    oh1 = eidx == i1
    p2 = jnp.where(oh1, -jnp.inf, prob)
    m2 = jnp.max(p2, axis=-1, keepdims=True)
    i2 = jnp.min(jnp.where(p2 == m2, eidx, E), axis=-1, keepdims=True)
    oh2 = eidx == i2
    denom = m1 + m2
    ew_ref[...] = (jnp.where(oh1, m1, 0.0) + jnp.where(oh2, m2, 0.0)) / denom


def _moe_body(sn_ref, d2_ref, ew_ref, a1_ref, a3_ref,
              w1_ref, w3_ref, w1lb_ref, w3lb_ref,
              w2_ref, w2la_ref, w2lb_ref,
              o_ref, acc_ref, u_ref):
    j = pl.program_id(1)

    @pl.when(j == 0)
    def _init():
        acc_ref[...] = jnp.zeros_like(acc_ref)
        u_ref[...] = jnp.zeros_like(u_ref)

    sn = sn_ref[...]
    cw1 = jnp.dot(sn, w1_ref[...], preferred_element_type=jnp.float32)
    cw3 = jnp.dot(sn, w3_ref[...], preferred_element_type=jnp.float32)
    ew = ew_ref[...]
    a1 = a1_ref[...]
    a3 = a3_ref[...]
    zsum = jnp.zeros((TS_E, F_E), jnp.float32)
    dus = []
    for e in range(E):
        d1 = jnp.dot(a1[:, e * R:(e + 1) * R], w1lb_ref[e],
                     preferred_element_type=jnp.float32)
        d3 = jnp.dot(a3[:, e * R:(e + 1) * R], w3lb_ref[e],
                     preferred_element_type=jnp.float32)
        lw1 = cw1 + d1 * SCALE
        lw3 = cw3 + d3 * SCALE
        s = lw1 * lax.logistic(lw1) * lw3
        z = ew[:, e:e + 1] * s
        zsum = zsum + z
        dus.append(jnp.dot(z, w2la_ref[e], preferred_element_type=jnp.float32))
    u_ref[...] += jnp.concatenate(dus, axis=1)
    acc_ref[...] += jnp.dot(zsum, w2_ref[...],
                            preferred_element_type=jnp.float32)

    @pl.when(j == NJ - 1)
    def _fin():
        lora2 = jnp.zeros((TS_E, D), jnp.float32)
        uu = u_ref[...]
        for e in range(E):
            lora2 = lora2 + jnp.dot(uu[:, e * R:(e + 1) * R], w2lb_ref[e],
                                    preferred_element_type=jnp.float32)
        o_ref[...] = acc_ref[...] + d2_ref[...] + lora2 * SCALE


def _run(data, rope_cos, rope_sin, wq, wk, wv, wo, w1, w2, w3,
         gate_w, attn_norm_w, ffn_norm_w, w1_la, w1_lb, w3_la, w3_lb,
         w2_la, w2_lb):
    f = jnp.float32
    x = data.reshape(S, D)

    def perm(w, nh):
        return w.reshape(D, nh, HD // 2, 2).transpose(0, 1, 3, 2).reshape(D, nh * HD)

    wqp = perm(wq, NH)
    wkp = perm(wk, NKV)
    wqkv = jnp.concatenate([wqp, wkp, wv], axis=1)
    cc = jnp.concatenate([rope_cos, rope_cos], axis=1)
    ss2 = jnp.concatenate([-rope_sin, rope_sin], axis=1)
    eye = jnp.eye(HD // 2, dtype=f)
    zz = jnp.zeros((HD // 2, HD // 2), f)
    pmat = jnp.block([[zz, eye], [eye, zz]])

    q, k, v = pl.pallas_call(
        _qkv_body,
        grid=(S // TS_A,),
        in_specs=[
            pl.BlockSpec((TS_A, D), lambda i: (i, 0)),
            pl.BlockSpec((1, D), lambda i: (0, 0)),
            pl.BlockSpec((D, (NH + 2 * NKV) * HD), lambda i: (0, 0)),
            pl.BlockSpec((TS_A, HD), lambda i: (i, 0)),
            pl.BlockSpec((TS_A, HD), lambda i: (i, 0)),
            pl.BlockSpec((HD, HD), lambda i: (0, 0)),
        ],
        out_specs=[
            pl.BlockSpec((TS_A, NH * HD), lambda i: (i, 0)),
            pl.BlockSpec((TS_A, NKV * HD), lambda i: (i, 0)),
            pl.BlockSpec((TS_A, NKV * HD), lambda i: (i, 0)),
        ],
        out_shape=[
            jax.ShapeDtypeStruct((S, NH * HD), f),
            jax.ShapeDtypeStruct((S, NKV * HD), f),
            jax.ShapeDtypeStruct((S, NKV * HD), f),
        ],
        compiler_params=pltpu.CompilerParams(
            dimension_semantics=("arbitrary",)),
    )(x, attn_norm_w.reshape(1, D), wqkv, cc, ss2, pmat)

    attn = pl.pallas_call(
        _attn_body,
        grid=(NH, S // TS_Q),
        in_specs=[
            pl.BlockSpec((TS_Q, HD), lambda h, i: (i, h)),
            pl.BlockSpec((S, HD), lambda h, i: (0, h // 2)),
            pl.BlockSpec((S, HD), lambda h, i: (0, h // 2)),
        ],
        out_specs=pl.BlockSpec((TS_Q, HD), lambda h, i: (i, h)),
        out_shape=jax.ShapeDtypeStruct((S, NH * HD), f),
        compiler_params=pltpu.CompilerParams(
            dimension_semantics=("arbitrary", "arbitrary")),
    )(q, k, v)

    la1 = w1_la.transpose(1, 0, 2).reshape(D, E * R)
    la3 = w3_la.transpose(1, 0, 2).reshape(D, E * R)

    d2, sn, ew, a1, a3 = pl.pallas_call(
        _proj_router_body,
        grid=(S // TS_C,),
        in_specs=[
            pl.BlockSpec((TS_C, NH * HD), lambda i: (i, 0)),
            pl.BlockSpec((TS_C, D), lambda i: (i, 0)),
            pl.BlockSpec((NH * HD, D), lambda i: (0, 0)),
            pl.BlockSpec((1, D), lambda i: (0, 0)),
            pl.BlockSpec((D, E), lambda i: (0, 0)),
            pl.BlockSpec((D, E * R), lambda i: (0, 0)),
            pl.BlockSpec((D, E * R), lambda i: (0, 0)),
        ],
        out_specs=[
            pl.BlockSpec((TS_C, D), lambda i: (i, 0)),
            pl.BlockSpec((TS_C, D), lambda i: (i, 0)),
            pl.BlockSpec((TS_C, E), lambda i: (i, 0)),
            pl.BlockSpec((TS_C, E * R), lambda i: (i, 0)),
            pl.BlockSpec((TS_C, E * R), lambda i: (i, 0)),
        ],
        out_shape=[
            jax.ShapeDtypeStruct((S, D), f),
            jax.ShapeDtypeStruct((S, D), f),
            jax.ShapeDtypeStruct((S, E), f),
            jax.ShapeDtypeStruct((S, E * R), f),
            jax.ShapeDtypeStruct((S, E * R), f),
        ],
        compiler_params=pltpu.CompilerParams(
            dimension_semantics=("arbitrary",)),
    )(attn, x, wo, ffn_norm_w.reshape(1, D), gate_w, la1, la3)

    out = pl.pallas_call(
        _moe_body,
        grid=(S // TS_E, NJ),
        in_specs=[
            pl.BlockSpec((TS_E, D), lambda s, j: (s, 0)),
            pl.BlockSpec((TS_E, D), lambda s, j: (s, 0)),
            pl.BlockSpec((TS_E, E), lambda s, j: (s, 0)),
            pl.BlockSpec((TS_E, E * R), lambda s, j: (s, 0)),
            pl.BlockSpec((TS_E, E * R), lambda s, j: (s, 0)),
            pl.BlockSpec((D, F_E), lambda s, j: (0, j)),
            pl.BlockSpec((D, F_E), lambda s, j: (0, j)),
            pl.BlockSpec((E, R, F_E), lambda s, j: (0, 0, j)),
            pl.BlockSpec((E, R, F_E), lambda s, j: (0, 0, j)),
            pl.BlockSpec((F_E, D), lambda s, j: (j, 0)),
            pl.BlockSpec((E, F_E, R), lambda s, j: (0, j, 0)),
            pl.BlockSpec((E, R, D), lambda s, j: (0, 0, 0)),
        ],
        out_specs=pl.BlockSpec((TS_E, D), lambda s, j: (s, 0)),
        out_shape=jax.ShapeDtypeStruct((S, D), f),
        scratch_shapes=[
            pltpu.VMEM((TS_E, D), f),
            pltpu.VMEM((TS_E, E * R), f),
        ],
        compiler_params=pltpu.CompilerParams(
            dimension_semantics=("arbitrary", "arbitrary")),
    )(sn, d2, ew, a1, a3, w1, w3, w1_lb, w3_lb, w2, w2_la, w2_lb)

    return out.reshape(B, S, D)


def kernel(data, mask, rope_cos, rope_sin, wq, wk, wv, wo, w1, w2, w3,
           gate_w, attn_norm_w, ffn_norm_w, w1_la, w1_lb, w3_la, w3_lb,
           w2_la, w2_lb):
    del mask
    return _run(data, rope_cos, rope_sin, wq, wk, wv, wo, w1, w2, w3,
                gate_w, attn_norm_w, ffn_norm_w, w1_la, w1_lb, w3_la,
                w3_lb, w2_la, w2_lb)


# speedup vs baseline: 2.4791x; 1.0699x over previous
import jax
import jax.numpy as jnp
from jax import lax
from jax.experimental import pallas as pl
from jax.experimental.pallas import tpu as pltpu

B, S, D = 1, 2048, 2048
NH, NKV = 16, 8
HD = D // NH
DFF = 5632
E, K = 8, 2
R = 16
SCALE = 32.0 / 16.0
EPS = 1e-5

TS_A = 256
TS_Q = 256
TS_C = 256
TS_E = 256
F_E = 512
NJ = DFF // F_E

BF = jnp.bfloat16
F32 = jnp.float32


def _dot(a, b):
    return jnp.dot(a, b, preferred_element_type=F32)


def _qkv_body(x_ref, nw_ref, wq_ref, wk_ref, wv_ref, cc_ref, ss_ref, p_ref,
              q_ref, k_ref, v_ref):
    x = x_ref[...]
    var = jnp.mean(x * x, axis=-1, keepdims=True)
    h = (x * lax.rsqrt(var + EPS) * nw_ref[...]).astype(BF)
    q = _dot(h, wq_ref[...])
    k = _dot(h, wk_ref[...])
    v_ref[...] = _dot(h, wv_ref[...]).astype(BF)
    cc = cc_ref[...]
    ss = ss_ref[...]
    p = p_ref[...]
    for hh in range(NH):
        qh = q[:, hh * HD:(hh + 1) * HD]
        sw = _dot(qh.astype(BF), p)
        q_ref[:, hh * HD:(hh + 1) * HD] = (qh * cc + sw * ss).astype(BF)
    for hh in range(NKV):
        kh = k[:, hh * HD:(hh + 1) * HD]
        sw = _dot(kh.astype(BF), p)
        k_ref[:, hh * HD:(hh + 1) * HD] = (kh * cc + sw * ss).astype(BF)


def _attn_body(q_ref, k_ref, v_ref, o_ref):
    i = pl.program_id(1)
    q = q_ref[...]
    k = k_ref[...]
    scores = lax.dot_general(q, k, (((1,), (1,)), ((), ())),
                             preferred_element_type=F32)
    scores = scores * (1.0 / (HD ** 0.5))
    row = lax.broadcasted_iota(jnp.int32, scores.shape, 0) + i * TS_Q
    col = lax.broadcasted_iota(jnp.int32, scores.shape, 1)
    scores = jnp.where(col <= row, scores, -1e9)
    m = jnp.max(scores, axis=-1, keepdims=True)
    p = jnp.exp(scores - m)
    att = (p / jnp.sum(p, axis=-1, keepdims=True)).astype(BF)
    o_ref[...] = lax.dot_general(att, v_ref[...], (((1,), (0,)), ((), ())),
                                 preferred_element_type=F32).astype(BF)


def _proj_router_body(attn_ref, data_ref, wo_ref, nw_ref, gw_ref,
                      la1_ref, la3_ref,
                      d2_ref, sn_ref, ew_ref, a1_ref, a3_ref):
    d2 = _dot(attn_ref[...], wo_ref[...]) + data_ref[...]
    d2_ref[...] = d2
    var = jnp.mean(d2 * d2, axis=-1, keepdims=True)
    sn = d2 * lax.rsqrt(var + EPS) * nw_ref[...]
    snb = sn.astype(BF)
    sn_ref[...] = snb
    a1_ref[...] = _dot(snb, la1_ref[...])
    a3_ref[...] = _dot(snb, la3_ref[...])
    logits = _dot(snb, gw_ref[...])
    mx = jnp.max(logits, axis=-1, keepdims=True)
    pexp = jnp.exp(logits - mx)
    prob = pexp / jnp.sum(pexp, axis=-1, keepdims=True)
    eidx = lax.broadcasted_iota(jnp.int32, prob.shape, 1)
    m1 = jnp.max(prob, axis=-1, keepdims=True)
    i1 = jnp.min(jnp.where(prob == m1, eidx, E), axis=-1, keepdims=True)
    oh1 = eidx == i1
    p2 = jnp.where(oh1, -jnp.inf, prob)
    m2 = jnp.max(p2, axis=-1, keepdims=True)
    i2 = jnp.min(jnp.where(p2 == m2, eidx, E), axis=-1, keepdims=True)
    oh2 = eidx == i2
    denom = m1 + m2
    ew_ref[...] = (jnp.where(oh1, m1, 0.0) + jnp.where(oh2, m2, 0.0)) / denom


def _moe_body(sn_ref, d2_ref, ew_ref, a1_ref, a3_ref,
              w1_ref, w3_ref, w1lb_ref, w3lb_ref,
              w2_ref, w2la_ref, w2lb_ref,
              o_ref, acc_ref, u_ref):
    j = pl.program_id(1)

    @pl.when(j == 0)
    def _init():
        acc_ref[...] = jnp.zeros_like(acc_ref)
        u_ref[...] = jnp.zeros_like(u_ref)

    sn = sn_ref[...]
    cw1 = _dot(sn, w1_ref[...])
    cw3 = _dot(sn, w3_ref[...])
    ew = ew_ref[...]
    a1 = a1_ref[...]
    a3 = a3_ref[...]
    zsum = jnp.zeros((TS_E, F_E), F32)
    dus = []
    for e in range(E):
        d1 = _dot(a1[:, e * R:(e + 1) * R].astype(BF), w1lb_ref[e])
        d3 = _dot(a3[:, e * R:(e + 1) * R].astype(BF), w3lb_ref[e])
        lw1 = cw1 + d1 * SCALE
        lw3 = cw3 + d3 * SCALE
        s = lw1 * lax.logistic(lw1) * lw3
        z = ew[:, e:e + 1] * s
        zsum = zsum + z
        dus.append(_dot(z.astype(BF), w2la_ref[e]))
    u_ref[...] += jnp.concatenate(dus, axis=1)
    acc_ref[...] += _dot(zsum.astype(BF), w2_ref[...])

    @pl.when(j == NJ - 1)
    def _fin():
        lora2 = jnp.zeros((TS_E, D), F32)
        uu = u_ref[...]
        for e in range(E):
            lora2 = lora2 + _dot(uu[:, e * R:(e + 1) * R].astype(BF),
                                 w2lb_ref[e])
        o_ref[...] = acc_ref[...] + d2_ref[...] + lora2 * SCALE


def _run(data, rope_cos, rope_sin, wq, wk, wv, wo, w1, w2, w3,
         gate_w, attn_norm_w, ffn_norm_w, w1_la, w1_lb, w3_la, w3_lb,
         w2_la, w2_lb):
    x = data.reshape(S, D)

    cc = jnp.stack([rope_cos, rope_cos], axis=-1).reshape(S, HD)
    ss = jnp.stack([-rope_sin, rope_sin], axis=-1).reshape(S, HD)
    ii = jnp.arange(HD)
    pmat = (ii[:, None] == (ii[None, :] ^ 1)).astype(BF)

    q, k, v = pl.pallas_call(
        _qkv_body,
        grid=(S // TS_A,),
        in_specs=[
            pl.BlockSpec((TS_A, D), lambda i: (i, 0)),
            pl.BlockSpec((1, D), lambda i: (0, 0)),
            pl.BlockSpec((D, NH * HD), lambda i: (0, 0)),
            pl.BlockSpec((D, NKV * HD), lambda i: (0, 0)),
            pl.BlockSpec((D, NKV * HD), lambda i: (0, 0)),
            pl.BlockSpec((TS_A, HD), lambda i: (i, 0)),
            pl.BlockSpec((TS_A, HD), lambda i: (i, 0)),
            pl.BlockSpec((HD, HD), lambda i: (0, 0)),
        ],
        out_specs=[
            pl.BlockSpec((TS_A, NH * HD), lambda i: (i, 0)),
            pl.BlockSpec((TS_A, NKV * HD), lambda i: (i, 0)),
            pl.BlockSpec((TS_A, NKV * HD), lambda i: (i, 0)),
        ],
        out_shape=[
            jax.ShapeDtypeStruct((S, NH * HD), BF),
            jax.ShapeDtypeStruct((S, NKV * HD), BF),
            jax.ShapeDtypeStruct((S, NKV * HD), BF),
        ],
        compiler_params=pltpu.CompilerParams(
            dimension_semantics=("arbitrary",)),
    )(x, attn_norm_w.reshape(1, D), wq.astype(BF), wk.astype(BF),
      wv.astype(BF), cc, ss, pmat)

    attn = pl.pallas_call(
        _attn_body,
        grid=(NH, S // TS_Q),
        in_specs=[
            pl.BlockSpec((TS_Q, HD), lambda h, i: (i, h)),
            pl.BlockSpec((S, HD), lambda h, i: (0, h // 2)),
            pl.BlockSpec((S, HD), lambda h, i: (0, h // 2)),
        ],
        out_specs=pl.BlockSpec((TS_Q, HD), lambda h, i: (i, h)),
        out_shape=jax.ShapeDtypeStruct((S, NH * HD), BF),
        compiler_params=pltpu.CompilerParams(
            dimension_semantics=("arbitrary", "arbitrary")),
    )(q, k, v)

    la1 = w1_la.transpose(1, 0, 2).reshape(D, E * R).astype(BF)
    la3 = w3_la.transpose(1, 0, 2).reshape(D, E * R).astype(BF)

    d2, sn, ew, a1, a3 = pl.pallas_call(
        _proj_router_body,
        grid=(S // TS_C,),
        in_specs=[
            pl.BlockSpec((TS_C, NH * HD), lambda i: (i, 0)),
            pl.BlockSpec((TS_C, D), lambda i: (i, 0)),
            pl.BlockSpec((NH * HD, D), lambda i: (0, 0)),
            pl.BlockSpec((1, D), lambda i: (0, 0)),
            pl.BlockSpec((D, E), lambda i: (0, 0)),
            pl.BlockSpec((D, E * R), lambda i: (0, 0)),
            pl.BlockSpec((D, E * R), lambda i: (0, 0)),
        ],
        out_specs=[
            pl.BlockSpec((TS_C, D), lambda i: (i, 0)),
            pl.BlockSpec((TS_C, D), lambda i: (i, 0)),
            pl.BlockSpec((TS_C, E), lambda i: (i, 0)),
            pl.BlockSpec((TS_C, E * R), lambda i: (i, 0)),
            pl.BlockSpec((TS_C, E * R), lambda i: (i, 0)),
        ],
        out_shape=[
            jax.ShapeDtypeStruct((S, D), F32),
            jax.ShapeDtypeStruct((S, D), BF),
            jax.ShapeDtypeStruct((S, E), F32),
            jax.ShapeDtypeStruct((S, E * R), F32),
            jax.ShapeDtypeStruct((S, E * R), F32),
        ],
        compiler_params=pltpu.CompilerParams(
            dimension_semantics=("arbitrary",)),
    )(attn, x, wo.astype(BF), ffn_norm_w.reshape(1, D), gate_w.astype(BF),
      la1, la3)

    out = pl.pallas_call(
        _moe_body,
        grid=(S // TS_E, NJ),
        in_specs=[
            pl.BlockSpec((TS_E, D), lambda s, j: (s, 0)),
            pl.BlockSpec((TS_E, D), lambda s, j: (s, 0)),
            pl.BlockSpec((TS_E, E), lambda s, j: (s, 0)),
            pl.BlockSpec((TS_E, E * R), lambda s, j: (s, 0)),
            pl.BlockSpec((TS_E, E * R), lambda s, j: (s, 0)),
            pl.BlockSpec((D, F_E), lambda s, j: (0, j)),
            pl.BlockSpec((D, F_E), lambda s, j: (0, j)),
            pl.BlockSpec((E, R, F_E), lambda s, j: (0, 0, j)),
            pl.BlockSpec((E, R, F_E), lambda s, j: (0, 0, j)),
            pl.BlockSpec((F_E, D), lambda s, j: (j, 0)),
            pl.BlockSpec((E, F_E, R), lambda s, j: (0, j, 0)),
            pl.BlockSpec((E, R, D), lambda s, j: (0, 0, 0)),
        ],
        out_specs=pl.BlockSpec((TS_E, D), lambda s, j: (s, 0)),
        out_shape=jax.ShapeDtypeStruct((S, D), F32),
        scratch_shapes=[
            pltpu.VMEM((TS_E, D), F32),
            pltpu.VMEM((TS_E, E * R), F32),
        ],
        compiler_params=pltpu.CompilerParams(
            dimension_semantics=("arbitrary", "arbitrary")),
    )(sn, d2, ew, a1, a3, w1.astype(BF), w3.astype(BF),
      w1_lb.astype(BF), w3_lb.astype(BF), w2.astype(BF),
      w2_la.astype(BF), w2_lb.astype(BF))

    return out.reshape(B, S, D)


def kernel(data, mask, rope_cos, rope_sin, wq, wk, wv, wo, w1, w2, w3,
           gate_w, attn_norm_w, ffn_norm_w, w1_la, w1_lb, w3_la, w3_lb,
           w2_la, w2_lb):
    del mask
    return _run(data, rope_cos, rope_sin, wq, wk, wv, wo, w1, w2, w3,
                gate_w, attn_norm_w, ffn_norm_w, w1_la, w1_lb, w3_la,
                w3_lb, w2_la, w2_lb)
